# baseline (speedup 1.0000x reference)
import functools

import jax
import jax.numpy as jnp
from jax import lax
from jax.experimental import pallas as pl
from jax.experimental.pallas import tpu as pltpu

_VMEM = 56 * 1024 * 1024


def _bn_relu_pool2(acc, s, b, H, W, C):
    Ho, Wo = H // 2, W // 2
    y = jnp.maximum(acc * s + b, 0.0)
    y3 = y.reshape(Ho, 2 * W, C)
    m = jnp.maximum(y3[:, :W, :], y3[:, W:, :]).reshape(Ho * Wo, 2, C)
    out = jnp.maximum(m[:, 0, :], m[:, 1, :])
    return out.astype(jnp.bfloat16)


def _conv3x3_wide(xf, wbig, H, W, Cin, Cout):
    HW = H * W
    zrow = jnp.zeros((W, Cin), xf.dtype)
    xp = jnp.concatenate([zrow, xf, zrow], axis=0)
    xcat = jnp.concatenate([xp[:HW], xf, xp[2 * W:]], axis=1)
    z = jnp.dot(xcat, wbig, preferred_element_type=jnp.float32)
    zl = z[:, :Cout]
    zc = z[:, Cout:2 * Cout]
    zr = z[:, 2 * Cout:]
    col = lax.broadcasted_iota(jnp.int32, (HW, 1), 0) & (W - 1)
    dn = jnp.concatenate([jnp.zeros((1, Cout), z.dtype), zl[:HW - 1]], axis=0)
    up = jnp.concatenate([zr[1:], jnp.zeros((1, Cout), z.dtype)], axis=0)
    acc = zc + jnp.where(col == 0, 0.0, dn) + jnp.where(col == W - 1, 0.0, up)
    return acc


def _tower_kernel(pe_ref, po_ref, w1_ref, s1_ref, b1_ref,
                  w2_ref, s2_ref, b2_ref, w3_ref, s3_ref, b3_ref,
                  w4_ref, s4_ref, b4_ref, w5_ref, s5_ref, b5_ref, o_ref):
    s1 = s1_ref[...]
    b1 = b1_ref[...]
    ae = jnp.dot(pe_ref[0], w1_ref[...], preferred_element_type=jnp.float32)
    ao = jnp.dot(po_ref[0], w1_ref[...], preferred_element_type=jnp.float32)
    m = jnp.maximum(jnp.maximum(ae * s1 + b1, 0.0),
                    jnp.maximum(ao * s1 + b1, 0.0))
    m3 = m.reshape(64, 128, 64)
    xq = (jnp.maximum(m3[:, :64, :], m3[:, 64:, :])
          .reshape(4096, 64).astype(jnp.bfloat16))
    a = _conv3x3_wide(xq, w2_ref[...], 64, 64, 64, 128)
    xq = _bn_relu_pool2(a, s2_ref[...], b2_ref[...], 64, 64, 128)
    a = _conv3x3_wide(xq, w3_ref[...], 32, 32, 128, 256)
    xq = _bn_relu_pool2(a, s3_ref[...], b3_ref[...], 32, 32, 256)
    a = _conv3x3_wide(xq, w4_ref[...], 16, 16, 256, 512)
    xq = _bn_relu_pool2(a, s4_ref[...], b4_ref[...], 16, 16, 512)
    a = _conv3x3_wide(xq, w5_ref[...], 8, 8, 512, 512)
    xq = _bn_relu_pool2(a, s5_ref[...], b5_ref[...], 8, 8, 512)
    o_ref[0] = xq


def _head_kernel(x_ref, w0_ref, b0_ref, w1_ref, b1_ref, w2_ref, b2_ref,
                 o_ref):
    h = jnp.dot(x_ref[...], w0_ref[...], preferred_element_type=jnp.float32)
    h = jnp.maximum(h + b0_ref[...], 0.0).astype(jnp.bfloat16)
    h = jnp.dot(h, w1_ref[...], preferred_element_type=jnp.float32)
    h = jnp.maximum(h + b1_ref[...], 0.0).astype(jnp.bfloat16)
    o_ref[...] = (jnp.dot(h, w2_ref[...], preferred_element_type=jnp.float32)
                  + b2_ref[...])


def _patches27_eo(x):
    N, H, W, C = x.shape
    xp = jnp.pad(x, ((0, 0), (1, 1), (1, 1), (0, 0)))
    outs = []
    for base in (0, 1):
        taps = [xp[:, dy:dy + H, dx + base::2, :][:, :, :W // 2, :]
                for dy in range(3) for dx in range(3)]
        outs.append(jnp.concatenate(taps, axis=-1)
                    .reshape(N, H * W // 2, 9 * C))
    return outs


def _mix_w(w, cin, cout):
    return (w.reshape(3, 3, cin, cout).transpose(0, 2, 1, 3)
            .reshape(3 * cin, 3 * cout))


def kernel(x, c0w, c0s, c0b, c1w, c1s, c1b, c2w, c2s, c2b, c3w, c3s, c3b,
           c4w, c4s, c4b, f0w, f0b, f1w, f1b, f2w, f2b):
    N = x.shape[0]
    xh = jnp.transpose(x, (0, 2, 3, 1)).astype(jnp.bfloat16)
    pe, po = _patches27_eo(xh)
    wb2 = _mix_w(c1w, 64, 128)
    wb3 = _mix_w(c2w, 128, 256)
    wb4 = _mix_w(c3w, 256, 512)
    wb5 = _mix_w(c4w, 512, 512)

    def _svec(s):
        return s.reshape(1, -1)

    wspec = lambda shape: pl.BlockSpec(shape, lambda n: (0,) * len(shape))
    tower = pl.pallas_call(
        _tower_kernel,
        out_shape=jax.ShapeDtypeStruct((N, 16, 512), jnp.bfloat16),
        grid=(N,),
        in_specs=[
            pl.BlockSpec((1, 8192, 27), lambda n: (n, 0, 0)),
            pl.BlockSpec((1, 8192, 27), lambda n: (n, 0, 0)),
            wspec((27, 64)), wspec((1, 64)), wspec((1, 64)),
            wspec((192, 384)), wspec((1, 128)), wspec((1, 128)),
            wspec((384, 768)), wspec((1, 256)), wspec((1, 256)),
            wspec((768, 1536)), wspec((1, 512)), wspec((1, 512)),
            wspec((1536, 1536)), wspec((1, 512)), wspec((1, 512)),
        ],
        out_specs=pl.BlockSpec((1, 16, 512), lambda n: (n, 0, 0)),
        compiler_params=pltpu.CompilerParams(
            dimension_semantics=("parallel",),
            vmem_limit_bytes=_VMEM),
    )(pe, po, c0w, _svec(c0s), _svec(c0b),
      wb2, _svec(c1s), _svec(c1b), wb3, _svec(c2s), _svec(c2b),
      wb4, _svec(c3s), _svec(c3b), wb5, _svec(c4s), _svec(c4b))

    xf = jnp.transpose(tower, (0, 2, 1)).reshape(N, 8192).astype(jnp.bfloat16)

    w2p = jnp.pad(f2w, ((0, 0), (0, 128 - f2w.shape[1])))
    b2p = jnp.pad(f2b, (0, 128 - f2b.shape[0]))
    Mh = N // 2
    head = pl.pallas_call(
        _head_kernel,
        out_shape=jax.ShapeDtypeStruct((N, 128), jnp.float32),
        grid=(2,),
        in_specs=[
            pl.BlockSpec((Mh, 8192), lambda m: (m, 0)),
            wspec((8192, 1024)), wspec((1, 1024)),
            wspec((1024, 512)), wspec((1, 512)),
            wspec((512, 128)), wspec((1, 128)),
        ],
        out_specs=pl.BlockSpec((Mh, 128), lambda m: (m, 0)),
        compiler_params=pltpu.CompilerParams(
            dimension_semantics=("parallel",),
            vmem_limit_bytes=_VMEM),
    )(xf, f0w, _svec(f0b), f1w, _svec(f1b), w2p, _svec(b2p))
    return head[:, :11]


# speedup vs baseline: 3.2065x; 3.2065x over previous
import functools

import jax
import jax.numpy as jnp
from jax import lax
from jax.experimental import pallas as pl
from jax.experimental.pallas import tpu as pltpu

_VMEM = 56 * 1024 * 1024


def _bn_relu_pool2(acc, s, b, H, W, C):
    Ho, Wo = H // 2, W // 2
    y = jnp.maximum(acc * s + b, 0.0)
    y3 = y.reshape(Ho, 2 * W, C)
    m = jnp.maximum(y3[:, :W, :], y3[:, W:, :]).reshape(Ho * Wo, 2, C)
    out = jnp.maximum(m[:, 0, :], m[:, 1, :])
    return out.astype(jnp.bfloat16)


def _conv3x3_wide(xf, wbig, H, W, Cin, Cout):
    HW = H * W
    zrow = jnp.zeros((W, Cin), xf.dtype)
    xp = jnp.concatenate([zrow, xf, zrow], axis=0)
    xcat = jnp.concatenate([xp[:HW], xf, xp[2 * W:]], axis=1)
    z = jnp.dot(xcat, wbig, preferred_element_type=jnp.float32)
    zl = z[:, :Cout]
    zc = z[:, Cout:2 * Cout]
    zr = z[:, 2 * Cout:]
    col = lax.broadcasted_iota(jnp.int32, (HW, 1), 0) & (W - 1)
    dn = jnp.concatenate([jnp.zeros((1, Cout), z.dtype), zl[:HW - 1]], axis=0)
    up = jnp.concatenate([zr[1:], jnp.zeros((1, Cout), z.dtype)], axis=0)
    acc = zc + jnp.where(col == 0, 0.0, dn) + jnp.where(col == W - 1, 0.0, up)
    return acc


def _tower_kernel(p_ref, w1_ref, s1_ref, b1_ref,
                  w2_ref, s2_ref, b2_ref, w3_ref, s3_ref, b3_ref,
                  w4_ref, s4_ref, b4_ref, w5_ref, s5_ref, b5_ref, o_ref):
    a = jnp.dot(p_ref[0], w1_ref[...], preferred_element_type=jnp.float32)
    xq = _bn_relu_pool2(a, s1_ref[...], b1_ref[...], 128, 128, 64)
    a = _conv3x3_wide(xq, w2_ref[...], 64, 64, 64, 128)
    xq = _bn_relu_pool2(a, s2_ref[...], b2_ref[...], 64, 64, 128)
    a = _conv3x3_wide(xq, w3_ref[...], 32, 32, 128, 256)
    xq = _bn_relu_pool2(a, s3_ref[...], b3_ref[...], 32, 32, 256)
    a = _conv3x3_wide(xq, w4_ref[...], 16, 16, 256, 512)
    xq = _bn_relu_pool2(a, s4_ref[...], b4_ref[...], 16, 16, 512)
    a = _conv3x3_wide(xq, w5_ref[...], 8, 8, 512, 512)
    xq = _bn_relu_pool2(a, s5_ref[...], b5_ref[...], 8, 8, 512)
    o_ref[0] = xq


def _head_kernel(x_ref, w0_ref, b0_ref, w1_ref, b1_ref, w2_ref, b2_ref,
                 o_ref):
    h = jnp.dot(x_ref[...], w0_ref[...], preferred_element_type=jnp.float32)
    h = jnp.maximum(h + b0_ref[...], 0.0).astype(jnp.bfloat16)
    h = jnp.dot(h, w1_ref[...], preferred_element_type=jnp.float32)
    h = jnp.maximum(h + b1_ref[...], 0.0).astype(jnp.bfloat16)
    o_ref[...] = (jnp.dot(h, w2_ref[...], preferred_element_type=jnp.float32)
                  + b2_ref[...])


def _patches27(x):
    N, H, W, C = x.shape
    xp = jnp.pad(x, ((0, 0), (1, 1), (1, 1), (0, 0)))
    taps = [xp[:, dy:dy + H, dx:dx + W, :]
            for dy in range(3) for dx in range(3)]
    return jnp.concatenate(taps, axis=-1).reshape(N, H * W, 9 * C)


def _mix_w(w, cin, cout):
    return (w.reshape(3, 3, cin, cout).transpose(0, 2, 1, 3)
            .reshape(3 * cin, 3 * cout))


def kernel(x, c0w, c0s, c0b, c1w, c1s, c1b, c2w, c2s, c2b, c3w, c3s, c3b,
           c4w, c4s, c4b, f0w, f0b, f1w, f1b, f2w, f2b):
    N = x.shape[0]
    xh = jnp.transpose(x, (0, 2, 3, 1)).astype(jnp.bfloat16)
    patches = _patches27(xh)
    wb2 = _mix_w(c1w, 64, 128)
    wb3 = _mix_w(c2w, 128, 256)
    wb4 = _mix_w(c3w, 256, 512)
    wb5 = _mix_w(c4w, 512, 512)

    def _svec(s):
        return s.reshape(1, -1)

    wspec = lambda shape: pl.BlockSpec(shape, lambda n: (0,) * len(shape))
    tower = pl.pallas_call(
        _tower_kernel,
        out_shape=jax.ShapeDtypeStruct((N, 16, 512), jnp.bfloat16),
        grid=(N,),
        in_specs=[
            pl.BlockSpec((1, 16384, 27), lambda n: (n, 0, 0)),
            wspec((27, 64)), wspec((1, 64)), wspec((1, 64)),
            wspec((192, 384)), wspec((1, 128)), wspec((1, 128)),
            wspec((384, 768)), wspec((1, 256)), wspec((1, 256)),
            wspec((768, 1536)), wspec((1, 512)), wspec((1, 512)),
            wspec((1536, 1536)), wspec((1, 512)), wspec((1, 512)),
        ],
        out_specs=pl.BlockSpec((1, 16, 512), lambda n: (n, 0, 0)),
        compiler_params=pltpu.CompilerParams(
            dimension_semantics=("parallel",),
            vmem_limit_bytes=_VMEM),
    )(patches, c0w, _svec(c0s), _svec(c0b),
      wb2, _svec(c1s), _svec(c1b), wb3, _svec(c2s), _svec(c2b),
      wb4, _svec(c3s), _svec(c3b), wb5, _svec(c4s), _svec(c4b))

    xf = jnp.transpose(tower, (0, 2, 1)).reshape(N, 8192).astype(jnp.bfloat16)

    w2p = jnp.pad(f2w, ((0, 0), (0, 128 - f2w.shape[1])))
    b2p = jnp.pad(f2b, (0, 128 - f2b.shape[0]))
    Mh = N // 2
    head = pl.pallas_call(
        _head_kernel,
        out_shape=jax.ShapeDtypeStruct((N, 128), jnp.float32),
        grid=(2,),
        in_specs=[
            pl.BlockSpec((Mh, 8192), lambda m: (m, 0)),
            wspec((8192, 1024)), wspec((1, 1024)),
            wspec((1024, 512)), wspec((1, 512)),
            wspec((512, 128)), wspec((1, 128)),
        ],
        out_specs=pl.BlockSpec((Mh, 128), lambda m: (m, 0)),
        compiler_params=pltpu.CompilerParams(
            dimension_semantics=("parallel",),
            vmem_limit_bytes=_VMEM),
    )(xf, f0w, _svec(f0b), f1w, _svec(f1b), w2p, _svec(b2p))
    return head[:, :11]
